# baseline (speedup 1.0000x reference)
import functools

import jax
import jax.numpy as jnp
from jax import lax
from jax.experimental import pallas as pl
from jax.experimental.pallas import tpu as pltpu
from jax.experimental.pallas import tpu_sc as plsc

N_NODES = 10000
N_EDGES = 160000
D_IN = 256
D_OUT = 256
DH = D_IN // 2
NS = 16
EPS = N_EDGES // NS
CH = 80
NCHUNK = EPS // CH
NBUF = 4
NGRP = (NCHUNK - 1) // NBUF
ROWS0 = 624
ROWS_LAST = N_NODES - 15 * ROWS0

_mesh = plsc.VectorSubcoreMesh(core_axis_name="c", subcore_axis_name="s")

_scratch = (
    [pltpu.VMEM((1, CH), jnp.int32) for _ in range(NBUF)]
    + [pltpu.VMEM((1, CH), jnp.int32) for _ in range(NBUF)]
    + [pltpu.VMEM((CH, DH), jnp.float32) for _ in range(NBUF)]
    + [pltpu.VMEM_SHARED((N_NODES, DH), jnp.float32)]
    + [pltpu.SemaphoreType.DMA] * (3 * NBUF)
)


@functools.partial(
    pl.kernel,
    mesh=_mesh,
    out_type=(
        jax.ShapeDtypeStruct((N_NODES, DH), jnp.float32),
        jax.ShapeDtypeStruct((N_NODES, DH), jnp.float32),
    ),
    scratch_types=_scratch,
)
def _scatter_sum(x_l, x_r, edges, zeros, h_l, h_r, *refs):
    srcb = refs[0:NBUF]
    dstb = refs[NBUF:2 * NBUF]
    rows = refs[2 * NBUF:3 * NBUF]
    h_sh = refs[3 * NBUF]
    isem = refs[3 * NBUF + 1:3 * NBUF + 1 + NBUF]
    gsem = refs[3 * NBUF + 1 + NBUF:3 * NBUF + 1 + 2 * NBUF]
    ssem = refs[3 * NBUF + 1 + 2 * NBUF:3 * NBUF + 1 + 3 * NBUF]

    c = lax.axis_index("c")
    s = lax.axis_index("s")
    base = pl.multiple_of(s * ROWS0, 8)

    @pl.when(s < 15)
    def _():
        pltpu.sync_copy(zeros.at[pl.ds(0, ROWS0)],
                        h_sh.at[pl.ds(base, ROWS0)])

    @pl.when(s == 15)
    def _():
        pltpu.sync_copy(zeros, h_sh.at[pl.ds(15 * ROWS0, ROWS_LAST)])

    plsc.subcore_barrier()

    def issue_idx(b, j):
        pltpu.async_copy(edges.at[0, s, j], srcb[b], isem[b])
        pltpu.async_copy(edges.at[1, s, j], dstb[b], isem[b])

    def wait_idx(b):
        pltpu.make_async_copy(edges.at[0, 0, 0], srcb[b], isem[b]).wait()
        pltpu.make_async_copy(edges.at[0, 0, 0], dstb[b], isem[b]).wait()

    def wait_gather(x_hbm, b):
        pltpu.make_async_copy(x_hbm.at[pl.ds(0, CH)], rows[b], gsem[b]).wait()

    def wait_scatter(x_hbm, b):
        pltpu.make_async_copy(x_hbm.at[pl.ds(0, CH)], rows[b], ssem[b]).wait()

    def _run(x_hbm):
        for b in range(NBUF):
            issue_idx(b, b)

        def body(g, carry):
            jg = NBUF * g
            for b in range(NBUF):
                wait_idx(b)
                pltpu.async_copy(x_hbm.at[srcb[b].at[0]], rows[b], gsem[b])
            for b in range(NBUF):
                wait_gather(x_hbm, b)
                pltpu.async_copy(rows[b], h_sh.at[dstb[b].at[0]], ssem[b],
                                 add=True)
            for b in range(NBUF):
                wait_scatter(x_hbm, b)

                @pl.when(jg + NBUF + b < NCHUNK)
                def _(b=b):
                    issue_idx(b, jg + NBUF + b)
            return carry

        lax.fori_loop(0, NGRP, body, 0)

        wait_idx(0)
        pltpu.async_copy(x_hbm.at[srcb[0].at[0]], rows[0], gsem[0])
        wait_gather(x_hbm, 0)
        pltpu.async_copy(rows[0], h_sh.at[dstb[0].at[0]], ssem[0], add=True)
        wait_scatter(x_hbm, 0)

    @pl.when(c == 0)
    def _():
        _run(x_l)

    @pl.when(c == 1)
    def _():
        _run(x_r)

    plsc.subcore_barrier()

    h_out = [h_l, h_r]
    for ci, h_hbm in enumerate(h_out):
        @pl.when((c == ci) & (s < 15))
        def _(h_hbm=h_hbm):
            row = pl.ds(base, ROWS0)
            pltpu.sync_copy(h_sh.at[row], h_hbm.at[row])

        @pl.when((c == ci) & (s == 15))
        def _(h_hbm=h_hbm):
            row = pl.ds(15 * ROWS0, ROWS_LAST)
            pltpu.sync_copy(h_sh.at[row], h_hbm.at[row])


BR = 1000


def _mm_body(hl_ref, hr_ref, wl_ref, wr_ref, b_ref, o_ref):
    acc = lax.dot_general(hl_ref[...], wl_ref[...],
                          (((1,), (1,)), ((), ())),
                          preferred_element_type=jnp.float32)
    acc = acc + lax.dot_general(hr_ref[...], wr_ref[...],
                                (((1,), (1,)), ((), ())),
                                preferred_element_type=jnp.float32)
    o_ref[...] = jnp.maximum(acc + b_ref[...], 0.0)


_matmul = pl.pallas_call(
    _mm_body,
    grid=(N_NODES // BR,),
    in_specs=[
        pl.BlockSpec((BR, DH), lambda i: (i, 0)),
        pl.BlockSpec((BR, DH), lambda i: (i, 0)),
        pl.BlockSpec((D_OUT, DH), lambda i: (0, 0)),
        pl.BlockSpec((D_OUT, DH), lambda i: (0, 0)),
        pl.BlockSpec((1, D_OUT), lambda i: (0, 0)),
    ],
    out_specs=pl.BlockSpec((BR, D_OUT), lambda i: (i, 0)),
    out_shape=jax.ShapeDtypeStruct((N_NODES, D_OUT), jnp.float32),
)


def kernel(x, edge_index, W, b):
    x_l = x[:, :DH]
    x_r = x[:, DH:]
    edges = edge_index.reshape(2, NS, NCHUNK, 1, CH)
    zeros = jnp.zeros((ROWS_LAST, DH), jnp.float32)
    h_l, h_r = _scatter_sum(x_l, x_r, edges, zeros)
    w_l = W[:, :DH]
    w_r = W[:, DH:]
    return _matmul(h_l, h_r, w_l, w_r, b.reshape(1, D_OUT))
